# Initial kernel scaffold; baseline (speedup 1.0000x reference)
import functools

import jax
import jax.numpy as jnp
from jax import lax
from jax.experimental import pallas as pl
from jax.experimental.pallas import tpu as pltpu
from jax.experimental.pallas import tpu_sc as plsc

N = 8192
D = 64
B = 4096

_NC = 2
_NS = 16
_NW = _NC * _NS
_PB = B // _NW
_L = 16


def _sc_body(inp_hbm, outp_hbm, co_hbm, ie_hbm, ib_hbm, oe_hbm, ob_hbm,
             x_out, c_out, ieg_out, oeg_out,
             inp_v, outp_v, flat_v, x_v, ib_v, ob_v, ie_v, oe_v, sem):
    wid = lax.axis_index("s") * _NC + lax.axis_index("c")
    base = wid * _PB
    pltpu.sync_copy(inp_hbm.at[pl.ds(base, _PB)], inp_v)
    pltpu.sync_copy(outp_hbm.at[pl.ds(base, _PB)], outp_v)
    for j in range(_PB // _L):
        s = pl.ds(j * _L, _L)
        flat_v[s] = inp_v[s] * N + outp_v[s]
    cps = [
        pltpu.async_copy(co_hbm.at[flat_v], x_v, sem),
        pltpu.async_copy(ie_hbm.at[inp_v], ie_v, sem),
        pltpu.async_copy(oe_hbm.at[outp_v], oe_v, sem),
        pltpu.async_copy(ib_hbm.at[inp_v], ib_v, sem),
        pltpu.async_copy(ob_hbm.at[outp_v], ob_v, sem),
    ]
    for cp in cps:
        cp.wait()
    for j in range(_PB // _L):
        s = pl.ds(j * _L, _L)
        x_v[s] = x_v[s] + 1.0
        ib_v[s] = ib_v[s] + ob_v[s]
    pltpu.sync_copy(x_v, x_out.at[pl.ds(base, _PB)])
    pltpu.sync_copy(ib_v, c_out.at[pl.ds(base, _PB)])
    pltpu.sync_copy(ie_v, ieg_out.at[pl.ds(base, _PB)])
    pltpu.sync_copy(oe_v, oeg_out.at[pl.ds(base, _PB)])


_sc_gather = pl.kernel(
    _sc_body,
    out_type=(
        jax.ShapeDtypeStruct((B,), jnp.float32),
        jax.ShapeDtypeStruct((B,), jnp.float32),
        jax.ShapeDtypeStruct((B, D), jnp.float32),
        jax.ShapeDtypeStruct((B, D), jnp.float32),
    ),
    mesh=plsc.VectorSubcoreMesh(core_axis_name="c", subcore_axis_name="s"),
    scratch_types=[
        pltpu.VMEM((_PB,), jnp.int32),
        pltpu.VMEM((_PB,), jnp.int32),
        pltpu.VMEM((_PB,), jnp.int32),
        pltpu.VMEM((_PB,), jnp.float32),
        pltpu.VMEM((_PB,), jnp.float32),
        pltpu.VMEM((_PB,), jnp.float32),
        pltpu.VMEM((_PB, D), jnp.float32),
        pltpu.VMEM((_PB, D), jnp.float32),
        pltpu.SemaphoreType.DMA,
    ],
)


def _tc_body(x_ref, c_ref, ie_ref, oe_ref, out_ref):
    x = x_ref[:]
    c = c_ref[:]
    dot = jnp.sum(ie_ref[:] * oe_ref[:], axis=1)
    logx = jnp.log(x)
    w = jnp.where(x > 100.0, 1.0, jnp.exp(0.75 * jnp.log(x * 0.01)))
    d = dot - logx
    wd = w * d
    s1 = jnp.sum(wd * d)
    s2 = jnp.sum(wd)
    s3 = jnp.sum(w)
    c1 = jnp.sum(c)
    c2 = jnp.sum(c * c)
    out_ref[0, 0] = B * s1 + 2.0 * c1 * s2 + c2 * s3


_tc_reduce = pl.pallas_call(
    _tc_body,
    out_shape=jax.ShapeDtypeStruct((1, 1), jnp.float32),
    out_specs=pl.BlockSpec(memory_space=pltpu.SMEM),
)


def kernel(input, output, co_occur, in_embed, in_bias, out_embed, out_bias):
    co_flat = co_occur.reshape(N * N)
    x, c, ieg, oeg = _sc_gather(
        input.astype(jnp.int32), output.astype(jnp.int32), co_flat,
        in_embed, in_bias.reshape(N), out_embed, out_bias.reshape(N))
    res = _tc_reduce(x, c, ieg, oeg)
    return res[0, 0]


# speedup vs baseline: 1.0571x; 1.0571x over previous
import functools

import jax
import jax.numpy as jnp
from jax import lax
from jax.experimental import pallas as pl
from jax.experimental.pallas import tpu as pltpu
from jax.experimental.pallas import tpu_sc as plsc

N = 8192
D = 64
B = 4096

_NC = 2
_NS = 16
_NW = _NC * _NS
_PB = B // _NW
_L = 16


def _sc_body(inp_hbm, outp_hbm, co_hbm, ie_hbm, ib_hbm, oe_hbm, ob_hbm,
             x_out, c_out, ieg_out, oeg_out,
             inp_v, outp_v, flat_v, x_v, ib_v, ob_v, ie_v, oe_v, sem):
    wid = lax.axis_index("s") * _NC + lax.axis_index("c")
    base = wid * _PB
    pltpu.sync_copy(inp_hbm.at[pl.ds(base, _PB)], inp_v)
    pltpu.sync_copy(outp_hbm.at[pl.ds(base, _PB)], outp_v)
    for j in range(_PB // _L):
        s = pl.ds(j * _L, _L)
        flat_v[s] = inp_v[s] * N + outp_v[s]
    cps = [
        pltpu.async_copy(co_hbm.at[flat_v], x_v, sem),
        pltpu.async_copy(ie_hbm.at[inp_v], ie_v, sem),
        pltpu.async_copy(oe_hbm.at[outp_v], oe_v, sem),
        pltpu.async_copy(ib_hbm.at[inp_v], ib_v, sem),
        pltpu.async_copy(ob_hbm.at[outp_v], ob_v, sem),
    ]
    for cp in cps:
        cp.wait()
    for j in range(_PB // _L):
        s = pl.ds(j * _L, _L)
        x_v[s] = x_v[s] + 1.0
        ib_v[s] = ib_v[s] + ob_v[s]
    pltpu.sync_copy(x_v, x_out.at[pl.ds(base, _PB)])
    pltpu.sync_copy(ib_v, c_out.at[pl.ds(base, _PB)])
    pltpu.sync_copy(ie_v, ieg_out.at[pl.ds(base, _PB)])
    pltpu.sync_copy(oe_v, oeg_out.at[pl.ds(base, _PB)])


_sc_gather = pl.kernel(
    _sc_body,
    out_type=(
        jax.ShapeDtypeStruct((B,), jnp.float32),
        jax.ShapeDtypeStruct((B,), jnp.float32),
        jax.ShapeDtypeStruct((B, D), jnp.float32),
        jax.ShapeDtypeStruct((B, D), jnp.float32),
    ),
    mesh=plsc.VectorSubcoreMesh(core_axis_name="c", subcore_axis_name="s"),
    scratch_types=[
        pltpu.VMEM((_PB,), jnp.int32),
        pltpu.VMEM((_PB,), jnp.int32),
        pltpu.VMEM((_PB,), jnp.int32),
        pltpu.VMEM((_PB,), jnp.float32),
        pltpu.VMEM((_PB,), jnp.float32),
        pltpu.VMEM((_PB,), jnp.float32),
        pltpu.VMEM((_PB, D), jnp.float32),
        pltpu.VMEM((_PB, D), jnp.float32),
        pltpu.SemaphoreType.DMA,
    ],
    compiler_params=pltpu.CompilerParams(use_tc_tiling_on_sc=False),
)


def _tc_body(x_ref, c_ref, ie_ref, oe_ref, out_ref):
    x = x_ref[:]
    c = c_ref[:]
    dot = jnp.sum(ie_ref[:] * oe_ref[:], axis=1)
    logx = jnp.log(x)
    w = jnp.where(x > 100.0, 1.0, jnp.exp(0.75 * jnp.log(x * 0.01)))
    d = dot - logx
    wd = w * d
    s1 = jnp.sum(wd * d)
    s2 = jnp.sum(wd)
    s3 = jnp.sum(w)
    c1 = jnp.sum(c)
    c2 = jnp.sum(c * c)
    out_ref[0, 0] = B * s1 + 2.0 * c1 * s2 + c2 * s3


_tc_reduce = pl.pallas_call(
    _tc_body,
    out_shape=jax.ShapeDtypeStruct((1, 1), jnp.float32),
    out_specs=pl.BlockSpec(memory_space=pltpu.SMEM),
)


def kernel(input, output, co_occur, in_embed, in_bias, out_embed, out_bias):
    co_flat = co_occur.reshape(N * N)
    x, c, ieg, oeg = _sc_gather(
        input.astype(jnp.int32), output.astype(jnp.int32), co_flat,
        in_embed, in_bias.reshape(N), out_embed, out_bias.reshape(N))
    res = _tc_reduce(x, c, ieg, oeg)
    return res[0, 0]


# speedup vs baseline: 5.9455x; 5.6241x over previous
import functools

import jax
import jax.numpy as jnp
from jax import lax
from jax.experimental import pallas as pl
from jax.experimental.pallas import tpu as pltpu
from jax.experimental.pallas import tpu_sc as plsc

N = 8192
D = 64
B = 4096

_NC = 2
_NS = 16
_NW = _NC * _NS
_PB = B // _NW
_L = 16


def _sc_body(inp_hbm, outp_hbm, co_hbm, ie_hbm, ib_hbm, oe_hbm, ob_hbm,
             x_out, c_out, ieg_out, oeg_out,
             inp_v, outp_v, flat_v, x_v, ib_v, ob_v, ie_v, oe_v, sem):
    wid = lax.axis_index("s") * _NC + lax.axis_index("c")
    base = wid * _PB
    pltpu.sync_copy(inp_hbm.at[pl.ds(base, _PB)], inp_v)
    pltpu.sync_copy(outp_hbm.at[pl.ds(base, _PB)], outp_v)
    for j in range(_PB // _L):
        s = pl.ds(j * _L, _L)
        r = inp_v[s]
        c = outp_v[s]
        flat_v[s] = ((r >> 3) * 64 + (c >> 7)) * 1024 + (r & 7) * 128 + (c & 127)
    cps = [
        pltpu.async_copy(co_hbm.at[flat_v], x_v, sem),
        pltpu.async_copy(ie_hbm.at[inp_v], ie_v, sem),
        pltpu.async_copy(oe_hbm.at[outp_v], oe_v, sem),
        pltpu.async_copy(ib_hbm.at[inp_v], ib_v, sem),
        pltpu.async_copy(ob_hbm.at[outp_v], ob_v, sem),
    ]
    for cp in cps:
        cp.wait()
    for j in range(_PB // _L):
        s = pl.ds(j * _L, _L)
        x_v[s] = x_v[s] + 1.0
        ib_v[s] = ib_v[s] + ob_v[s]
    pltpu.sync_copy(x_v, x_out.at[pl.ds(base, _PB)])
    pltpu.sync_copy(ib_v, c_out.at[pl.ds(base, _PB)])
    pltpu.sync_copy(ie_v, ieg_out.at[pl.ds(base, _PB)])
    pltpu.sync_copy(oe_v, oeg_out.at[pl.ds(base, _PB)])


_sc_gather = pl.kernel(
    _sc_body,
    out_type=(
        jax.ShapeDtypeStruct((B,), jnp.float32),
        jax.ShapeDtypeStruct((B,), jnp.float32),
        jax.ShapeDtypeStruct((B, D), jnp.float32),
        jax.ShapeDtypeStruct((B, D), jnp.float32),
    ),
    mesh=plsc.VectorSubcoreMesh(core_axis_name="c", subcore_axis_name="s"),
    scratch_types=[
        pltpu.VMEM((_PB,), jnp.int32),
        pltpu.VMEM((_PB,), jnp.int32),
        pltpu.VMEM((_PB,), jnp.int32),
        pltpu.VMEM((_PB,), jnp.float32),
        pltpu.VMEM((_PB,), jnp.float32),
        pltpu.VMEM((_PB,), jnp.float32),
        pltpu.VMEM((_PB, D), jnp.float32),
        pltpu.VMEM((_PB, D), jnp.float32),
        pltpu.SemaphoreType.DMA,
    ],
    compiler_params=pltpu.CompilerParams(use_tc_tiling_on_sc=False),
)


def _tc_body(x_ref, c_ref, ie_ref, oe_ref, out_ref):
    x = x_ref[:]
    c = c_ref[:]
    dot = jnp.sum(ie_ref[:] * oe_ref[:], axis=1)
    logx = jnp.log(x)
    w = jnp.where(x > 100.0, 1.0, jnp.exp(0.75 * jnp.log(x * 0.01)))
    d = dot - logx
    wd = w * d
    s1 = jnp.sum(wd * d)
    s2 = jnp.sum(wd)
    s3 = jnp.sum(w)
    c1 = jnp.sum(c)
    c2 = jnp.sum(c * c)
    out_ref[0, 0] = B * s1 + 2.0 * c1 * s2 + c2 * s3


_tc_reduce = pl.pallas_call(
    _tc_body,
    out_shape=jax.ShapeDtypeStruct((1, 1), jnp.float32),
    out_specs=pl.BlockSpec(memory_space=pltpu.SMEM),
)


def kernel(input, output, co_occur, in_embed, in_bias, out_embed, out_bias):
    co_flat = (co_occur.reshape(N // 8, 8, N // 128, 128)
               .transpose(0, 2, 1, 3).reshape(N * N))
    x, c, ieg, oeg = _sc_gather(
        input.astype(jnp.int32), output.astype(jnp.int32), co_flat,
        in_embed, in_bias.reshape(N), out_embed, out_bias.reshape(N))
    res = _tc_reduce(x, c, ieg, oeg)
    return res[0, 0]
